# baseline (speedup 1.0000x reference)
import functools

import jax
import jax.numpy as jnp
from jax.experimental import pallas as pl
from jax.experimental.pallas import tpu as pltpu
from jax.experimental.pallas import tpu_sc as plsc

_HIGH = jax.lax.Precision.HIGHEST


def _patches1(x_chw):
    C, H, W = x_chw.shape
    OH = OW = H // 2
    xp = jnp.pad(x_chw, ((0, 0), (1, 3), (1, 3)))
    r = xp.reshape(C, OH + 2, 2, OW + 2, 2)
    ph = jnp.transpose(r, (0, 2, 4, 1, 3))
    ph = jax.lax.optimization_barrier(ph)
    cols = []
    for kh in range(4):
        for kw in range(4):
            for c in range(C):
                v = jax.lax.slice(
                    ph, (c, kh % 2, kw % 2, kh // 2, kw // 2),
                    (c + 1, kh % 2 + 1, kw % 2 + 1, kh // 2 + OH, kw // 2 + OW))
                cols.append(v.reshape(OH * OW))
    return jnp.stack(cols, axis=1).astype(jnp.bfloat16)


def _patches_s2(h_flat, H, C, k):
    OH = OW = H // 2
    hp = jnp.pad(h_flat.reshape(H, H, C), ((1, 3), (1, 3), (0, 0)))
    r = hp.reshape(OH + 2, 2, OW + 2, 2, C)
    ph = jnp.transpose(r, (1, 3, 0, 2, 4))
    parts = []
    for kh in range(k):
        for kw in range(k):
            v = jax.lax.slice(
                ph, (kh % 2, kw % 2, kh // 2, kw // 2, 0),
                (kh % 2 + 1, kw % 2 + 1, kh // 2 + OH, kw // 2 + OW, C))
            parts.append(v.reshape(OH * OW, C))
    return jnp.concatenate(parts, axis=1)


def _patches_s1(h_flat, H, C, k):
    hp = jnp.pad(h_flat.reshape(H, H, C), ((1, 1), (1, 1), (0, 0)))
    parts = []
    for kh in range(k):
        for kw in range(k):
            v = jax.lax.slice(hp, (kh, kw, 0), (kh + H, kw + H, C))
            parts.append(v.reshape(H * H, C))
    return jnp.concatenate(parts, axis=1)


def _mm_bias_kernel(x_ref, w_ref, b_ref, o_ref, *, relu):
    y = jnp.dot(x_ref[...], w_ref[...],
                preferred_element_type=jnp.float32)
    y = y + b_ref[...]
    if relu:
        y = jnp.maximum(y, 0.0)
    o_ref[...] = y.astype(jnp.bfloat16)


def _mm_bias(x, w, b, relu, bm):
    M, K = x.shape
    N = w.shape[1]
    return pl.pallas_call(
        functools.partial(_mm_bias_kernel, relu=relu),
        grid=(M // bm,),
        in_specs=[
            pl.BlockSpec((bm, K), lambda i: (i, 0)),
            pl.BlockSpec((K, N), lambda i: (0, 0)),
            pl.BlockSpec((1, N), lambda i: (0, 0)),
        ],
        out_specs=pl.BlockSpec((bm, N), lambda i: (i, 0)),
        out_shape=jax.ShapeDtypeStruct((M, N), jnp.bfloat16),
    )(x, w.astype(jnp.bfloat16), b.reshape(1, N))


_VQ_KC = 2048


def _conv2_kernel(h1p_ref, w_ref, b_ref, o_ref):
    i = pl.program_id(0)
    parts = []
    for kh in range(4):
        blk = h1p_ref[pl.ds(32 * i + kh, 32), :, :]
        ev = blk.reshape(16, 2, 97, 128)[:, 0]
        for kw2 in range(2):
            v = jax.lax.slice(ev, (0, kw2, 0), (16, kw2 + 96, 128))
            parts.append(v.reshape(16 * 96, 128))
    patches = jnp.concatenate(parts, axis=1)
    y = jnp.dot(patches, w_ref[...], preferred_element_type=jnp.float32)
    y = jnp.maximum(y + b_ref[...], 0.0)
    o_ref[...] = y.astype(jnp.bfloat16)


def _conv2_fused(h1, W2m, b2):
    h1p = jnp.pad(h1.reshape(192, 192, 64), ((1, 1), (1, 1), (0, 0)))
    h1p = h1p.reshape(194, 97, 128)
    return pl.pallas_call(
        _conv2_kernel,
        grid=(6,),
        in_specs=[
            pl.BlockSpec((194, 97, 128), lambda i: (0, 0, 0)),
            pl.BlockSpec((1024, 64), lambda i: (0, 0)),
            pl.BlockSpec((1, 64), lambda i: (0, 0)),
        ],
        out_specs=pl.BlockSpec((1536, 64), lambda i: (i, 0)),
        out_shape=jax.ShapeDtypeStruct((9216, 64), jnp.bfloat16),
    )(h1p, W2m.astype(jnp.bfloat16), b2.reshape(1, 64))


def _vq_kernel(h2p_ref, w3_ref, b3_ref, cbt_ref, idx_ref, *, rows):
    i = pl.program_id(0)
    parts = []
    for kh in range(3):
        blk = h2p_ref[pl.ds(rows * i + kh, rows), :, :]
        for kw in range(3):
            v = jax.lax.slice(blk, (0, kw, 0), (rows, kw + 96, 64))
            parts.append(v.reshape(rows * 96, 64))
    p3 = jnp.concatenate(parts, axis=1)
    z = jnp.dot(p3, w3_ref[...],
                preferred_element_type=jnp.float32)
    z = z + b3_ref[...]
    z2 = jnp.sum(z * z, axis=1, keepdims=True)
    zb = z.astype(jnp.bfloat16)
    bm = z.shape[0]
    K = cbt_ref.shape[1]
    lane_iota = jax.lax.broadcasted_iota(jnp.int32, (bm, 128), 1)

    def chunk(j, carry):
        acc_v, acc_i = carry
        cbt = cbt_ref[:, pl.ds(j * _VQ_KC, _VQ_KC)]
        cn = jnp.sum(cbt * cbt, axis=0, keepdims=True)
        s = (z2 + cn) - 2.0 * jnp.dot(zb, cbt.astype(jnp.bfloat16),
                                      preferred_element_type=jnp.float32)
        for i in range(_VQ_KC // 128):
            sl = jax.lax.slice(s, (0, i * 128), (bm, (i + 1) * 128))
            cand = lane_iota + (j * _VQ_KC + i * 128)
            upd = sl < acc_v
            acc_v = jnp.where(upd, sl, acc_v)
            acc_i = jnp.where(upd, cand, acc_i)
        return acc_v, acc_i

    acc_v0 = jnp.full((bm, 128), jnp.inf, jnp.float32)
    acc_i0 = jnp.zeros((bm, 128), jnp.int32)
    acc_v, acc_i = jax.lax.fori_loop(0, K // _VQ_KC, chunk, (acc_v0, acc_i0))
    m = jnp.min(acc_v, axis=1, keepdims=True)
    ii = jnp.where(acc_v == m, acc_i, jnp.int32(2**30))
    idx_ref[0, 0, :] = jnp.min(ii, axis=1)


def _vq_argmin(h2, w3, b3, codebook, rows):
    D = w3.shape[1]
    Kc = codebook.shape[0]
    nb = 96 // rows
    bm = rows * 96
    h2p = jnp.pad(h2.reshape(96, 96, 64), ((1, 1), (1, 1), (0, 0)))
    idx3 = pl.pallas_call(
        functools.partial(_vq_kernel, rows=rows),
        grid=(nb,),
        in_specs=[
            pl.BlockSpec((98, 98, 64), lambda i: (0, 0, 0)),
            pl.BlockSpec((576, D), lambda i: (0, 0)),
            pl.BlockSpec((1, D), lambda i: (0, 0)),
            pl.BlockSpec((D, Kc), lambda i: (0, 0)),
        ],
        out_specs=pl.BlockSpec((1, 1, bm), lambda i: (i, 0, 0)),
        out_shape=jax.ShapeDtypeStruct((nb, 1, bm), jnp.int32),
    )(h2p, w3.astype(jnp.bfloat16), b3.reshape(1, D), codebook.T)
    return idx3.reshape(96 * 96)


_SC_WINDOW = 256


def _sc_gather(codebook, idx):
    n = idx.shape[0]
    D = codebook.shape[1]
    idx2 = idx.reshape(1, n)
    mesh = plsc.VectorSubcoreMesh(core_axis_name="core",
                                  subcore_axis_name="subcore")

    @functools.partial(
        pl.kernel,
        out_type=jax.ShapeDtypeStruct((n, D), codebook.dtype),
        mesh=mesh)
    def gather_kernel(cb_hbm, i_hbm, o_hbm):
        def body(i_vmem, o_vmem):
            pltpu.sync_copy(cb_hbm.at[i_vmem.at[0]], o_vmem)

        pltpu.emit_pipeline(
            body,
            grid=(n // _SC_WINDOW,),
            in_specs=[pl.BlockSpec((1, _SC_WINDOW), index_map=lambda i: (0, i))],
            out_specs=[pl.BlockSpec((_SC_WINDOW, D), index_map=lambda i: (i, 0))],
            core_axis_name=("core", "subcore"),
            dimension_semantics=(pltpu.PARALLEL,),
        )(i_hbm, o_hbm)

    return gather_kernel(codebook, idx2)


def kernel(prev_screen, curr_screen, W1, b1, W2, b2, W3, b3, codebook):
    h, w = curr_screen.shape[-2], curr_screen.shape[-1]
    x = jnp.reshape(jnp.concatenate((prev_screen, curr_screen), axis=0),
                    (2, h, w))
    x = 1.0 - x

    p1 = _patches1(x)
    h1 = _mm_bias(p1, W1.reshape(32, 64), b1, True, 4608)
    h2 = _conv2_fused(h1, W2.reshape(1024, 64), b2)
    idx = _vq_argmin(h2, W3.reshape(576, 32), b3, codebook, 8)
    cb_pad = jnp.pad(codebook, ((0, 0), (0, 96)))
    q = _sc_gather(cb_pad, idx)[:, :32]
    return q.T.reshape(1, -1)


# speedup vs baseline: 1.5211x; 1.5211x over previous
import functools

import jax
import jax.numpy as jnp
from jax.experimental import pallas as pl
from jax.experimental.pallas import tpu as pltpu
from jax.experimental.pallas import tpu_sc as plsc

_HIGH = jax.lax.Precision.HIGHEST


def _patches1(x_chw):
    C, H, W = x_chw.shape
    OH = OW = H // 2
    xp = jnp.pad(x_chw, ((0, 0), (1, 3), (1, 3)))
    r = xp.reshape(C, OH + 2, 2, OW + 2, 2)
    ph = jnp.transpose(r, (0, 2, 4, 1, 3))
    ph = jax.lax.optimization_barrier(ph)
    cols = []
    for kh in range(4):
        for kw in range(4):
            for c in range(C):
                v = jax.lax.slice(
                    ph, (c, kh % 2, kw % 2, kh // 2, kw // 2),
                    (c + 1, kh % 2 + 1, kw % 2 + 1, kh // 2 + OH, kw // 2 + OW))
                cols.append(v.reshape(OH * OW))
    return jnp.stack(cols, axis=1).astype(jnp.bfloat16)


def _patches_s2(h_flat, H, C, k):
    OH = OW = H // 2
    hp = jnp.pad(h_flat.reshape(H, H, C), ((1, 3), (1, 3), (0, 0)))
    r = hp.reshape(OH + 2, 2, OW + 2, 2, C)
    ph = jnp.transpose(r, (1, 3, 0, 2, 4))
    parts = []
    for kh in range(k):
        for kw in range(k):
            v = jax.lax.slice(
                ph, (kh % 2, kw % 2, kh // 2, kw // 2, 0),
                (kh % 2 + 1, kw % 2 + 1, kh // 2 + OH, kw // 2 + OW, C))
            parts.append(v.reshape(OH * OW, C))
    return jnp.concatenate(parts, axis=1)


def _patches_s1(h_flat, H, C, k):
    hp = jnp.pad(h_flat.reshape(H, H, C), ((1, 1), (1, 1), (0, 0)))
    parts = []
    for kh in range(k):
        for kw in range(k):
            v = jax.lax.slice(hp, (kh, kw, 0), (kh + H, kw + H, C))
            parts.append(v.reshape(H * H, C))
    return jnp.concatenate(parts, axis=1)


def _mm_bias_kernel(x_ref, w_ref, b_ref, o_ref, *, relu):
    y = jnp.dot(x_ref[...], w_ref[...],
                preferred_element_type=jnp.float32)
    y = y + b_ref[...]
    if relu:
        y = jnp.maximum(y, 0.0)
    o_ref[...] = y.astype(jnp.bfloat16)


def _mm_bias(x, w, b, relu, bm):
    M, K = x.shape
    N = w.shape[1]
    return pl.pallas_call(
        functools.partial(_mm_bias_kernel, relu=relu),
        grid=(M // bm,),
        in_specs=[
            pl.BlockSpec((bm, K), lambda i: (i, 0)),
            pl.BlockSpec((K, N), lambda i: (0, 0)),
            pl.BlockSpec((1, N), lambda i: (0, 0)),
        ],
        out_specs=pl.BlockSpec((bm, N), lambda i: (i, 0)),
        out_shape=jax.ShapeDtypeStruct((M, N), jnp.bfloat16),
    )(x, w.astype(jnp.bfloat16), b.reshape(1, N))


_VQ_KC = 2048


def _conv_s2_kernel(xp_ref, w_ref, b_ref, o_ref, *, ow, wcols):
    i = pl.program_id(0)
    parts = []
    for kh in range(4):
        blk = xp_ref[pl.ds(32 * i + kh, 32), :, :]
        ev = blk.reshape(16, 2, wcols, 128)[:, 0]
        for kw2 in range(2):
            v = jax.lax.slice(ev, (0, kw2, 0), (16, kw2 + ow, 128))
            parts.append(v.reshape(16 * ow, 128))
    patches = jnp.concatenate(parts, axis=1)
    y = jnp.dot(patches, w_ref[...], preferred_element_type=jnp.float32)
    y = jnp.maximum(y + b_ref[...], 0.0)
    o_ref[...] = y.astype(jnp.bfloat16)


def _conv_s2_fused(xp, wm, b, oh, ow):
    hp, wcols, _ = xp.shape
    bm = 16 * ow
    return pl.pallas_call(
        functools.partial(_conv_s2_kernel, ow=ow, wcols=wcols),
        grid=(oh // 16,),
        in_specs=[
            pl.BlockSpec((hp, wcols, 128), lambda i: (0, 0, 0)),
            pl.BlockSpec((1024, 64), lambda i: (0, 0)),
            pl.BlockSpec((1, 64), lambda i: (0, 0)),
        ],
        out_specs=pl.BlockSpec((bm, 64), lambda i: (i, 0)),
        out_shape=jax.ShapeDtypeStruct((oh * ow, 64), jnp.bfloat16),
    )(xp, wm.astype(jnp.bfloat16), b.reshape(1, 64))


def _conv2_fused(h1, W2m, b2):
    h1p = jnp.pad(h1.reshape(192, 192, 64), ((1, 3), (1, 1), (0, 0)))
    h1p = h1p.reshape(196, 97, 128)
    return _conv_s2_fused(h1p, W2m, b2, 96, 96)


def _conv1_fused(x_chw, W1, b1):
    xpad = jnp.pad(x_chw, ((0, 0), (1, 3), (1, 3)))
    t = jnp.transpose(xpad, (1, 2, 0))
    t = jnp.pad(t, ((0, 0), (0, 0), (0, 62)))
    xp = t.reshape(388, 194, 128).astype(jnp.bfloat16)
    w = jnp.pad(W1, ((0, 0), (0, 0), (0, 62), (0, 0)))
    return _conv_s2_fused(xp, w.reshape(1024, 64), b1, 192, 192)


def _vq_kernel(h2p_ref, w3_ref, b3_ref, cbt_ref, idx_ref, *, rows):
    i = pl.program_id(0)
    parts = []
    for kh in range(3):
        blk = h2p_ref[pl.ds(rows * i + kh, rows), :, :]
        for kw in range(3):
            v = jax.lax.slice(blk, (0, kw, 0), (rows, kw + 96, 64))
            parts.append(v.reshape(rows * 96, 64))
    p3 = jnp.concatenate(parts, axis=1)
    z = jnp.dot(p3, w3_ref[...],
                preferred_element_type=jnp.float32)
    z = z + b3_ref[...]
    z2 = jnp.sum(z * z, axis=1, keepdims=True)
    zb = z.astype(jnp.bfloat16)
    bm = z.shape[0]
    K = cbt_ref.shape[1]
    lane_iota = jax.lax.broadcasted_iota(jnp.int32, (bm, 128), 1)

    def chunk(j, carry):
        acc_v, acc_i = carry
        cbt = cbt_ref[:, pl.ds(j * _VQ_KC, _VQ_KC)]
        cn = jnp.sum(cbt * cbt, axis=0, keepdims=True)
        s = (z2 + cn) - 2.0 * jnp.dot(zb, cbt.astype(jnp.bfloat16),
                                      preferred_element_type=jnp.float32)
        for i in range(_VQ_KC // 128):
            sl = jax.lax.slice(s, (0, i * 128), (bm, (i + 1) * 128))
            cand = lane_iota + (j * _VQ_KC + i * 128)
            upd = sl < acc_v
            acc_v = jnp.where(upd, sl, acc_v)
            acc_i = jnp.where(upd, cand, acc_i)
        return acc_v, acc_i

    acc_v0 = jnp.full((bm, 128), jnp.inf, jnp.float32)
    acc_i0 = jnp.zeros((bm, 128), jnp.int32)
    acc_v, acc_i = jax.lax.fori_loop(0, K // _VQ_KC, chunk, (acc_v0, acc_i0))
    m = jnp.min(acc_v, axis=1, keepdims=True)
    ii = jnp.where(acc_v == m, acc_i, jnp.int32(2**30))
    idx_ref[0, 0, :] = jnp.min(ii, axis=1)


def _vq_argmin(h2, w3, b3, codebook, rows):
    D = w3.shape[1]
    Kc = codebook.shape[0]
    nb = 96 // rows
    bm = rows * 96
    h2p = jnp.pad(h2.reshape(96, 96, 64), ((1, 1), (1, 1), (0, 0)))
    idx3 = pl.pallas_call(
        functools.partial(_vq_kernel, rows=rows),
        grid=(nb,),
        in_specs=[
            pl.BlockSpec((98, 98, 64), lambda i: (0, 0, 0)),
            pl.BlockSpec((576, D), lambda i: (0, 0)),
            pl.BlockSpec((1, D), lambda i: (0, 0)),
            pl.BlockSpec((D, Kc), lambda i: (0, 0)),
        ],
        out_specs=pl.BlockSpec((1, 1, bm), lambda i: (i, 0, 0)),
        out_shape=jax.ShapeDtypeStruct((nb, 1, bm), jnp.int32),
    )(h2p, w3.astype(jnp.bfloat16), b3.reshape(1, D), codebook.T)
    return idx3.reshape(96 * 96)


_SC_WINDOW = 256


def _sc_gather(codebook, idx):
    n = idx.shape[0]
    D = codebook.shape[1]
    idx2 = idx.reshape(1, n)
    mesh = plsc.VectorSubcoreMesh(core_axis_name="core",
                                  subcore_axis_name="subcore")

    @functools.partial(
        pl.kernel,
        out_type=jax.ShapeDtypeStruct((n, D), codebook.dtype),
        mesh=mesh)
    def gather_kernel(cb_hbm, i_hbm, o_hbm):
        def body(i_vmem, o_vmem):
            pltpu.sync_copy(cb_hbm.at[i_vmem.at[0]], o_vmem)

        pltpu.emit_pipeline(
            body,
            grid=(n // _SC_WINDOW,),
            in_specs=[pl.BlockSpec((1, _SC_WINDOW), index_map=lambda i: (0, i))],
            out_specs=[pl.BlockSpec((_SC_WINDOW, D), index_map=lambda i: (i, 0))],
            core_axis_name=("core", "subcore"),
            dimension_semantics=(pltpu.PARALLEL,),
        )(i_hbm, o_hbm)

    return gather_kernel(codebook, idx2)


def kernel(prev_screen, curr_screen, W1, b1, W2, b2, W3, b3, codebook):
    h, w = curr_screen.shape[-2], curr_screen.shape[-1]
    x = jnp.reshape(jnp.concatenate((prev_screen, curr_screen), axis=0),
                    (2, h, w))
    x = 1.0 - x

    h1 = _conv1_fused(x, W1, b1)
    h2 = _conv2_fused(h1, W2.reshape(1024, 64), b2)
    idx = _vq_argmin(h2, W3.reshape(576, 32), b3, codebook, 8)
    cb_pad = jnp.pad(codebook, ((0, 0), (0, 96)))
    q = _sc_gather(cb_pad, idx)[:, :32]
    return q.T.reshape(1, -1)
